# baseline (speedup 1.0000x reference)
import jax
import jax.numpy as jnp
from jax import lax
from jax.experimental import pallas as pl
from jax.experimental.pallas import tpu as pltpu

N = 8192
D_IN = 2048
D_OUT = 2048
BLK = 512


def _fused_body(flag_ref, x_ref, wp_ref, bp_ref, wk_ref, bk_ref, out_ref):
    xb = x_ref[...].astype(jnp.bfloat16)
    op = lax.dot_general(xb, wp_ref[...], (((1,), (1,)), ((), ())),
                         preferred_element_type=jnp.float32) + bp_ref[...]
    ok = lax.dot_general(xb, wk_ref[...], (((1,), (1,)), ((), ())),
                         preferred_element_type=jnp.float32) + bk_ref[...]
    mask = flag_ref[...] == 1
    out_ref[...] = jnp.where(mask, op, ok)


def kernel(x, is_pc2, W_pc2, b_pc2, W_ko, b_ko):
    flags = is_pc2.reshape(N, 1)
    wp = W_pc2.astype(jnp.bfloat16)
    wk = W_ko.astype(jnp.bfloat16)
    bp = b_pc2.reshape(1, D_OUT)
    bk = b_ko.reshape(1, D_OUT)
    grid = (N // BLK,)
    return pl.pallas_call(
        _fused_body,
        grid=grid,
        in_specs=[
            pl.BlockSpec((BLK, 1), lambda i: (i, 0)),
            pl.BlockSpec((BLK, D_IN), lambda i: (i, 0)),
            pl.BlockSpec((D_OUT, D_IN), lambda i: (0, 0)),
            pl.BlockSpec((1, D_OUT), lambda i: (0, 0)),
            pl.BlockSpec((D_OUT, D_IN), lambda i: (0, 0)),
            pl.BlockSpec((1, D_OUT), lambda i: (0, 0)),
        ],
        out_specs=pl.BlockSpec((BLK, D_OUT), lambda i: (i, 0)),
        out_shape=jax.ShapeDtypeStruct((N, D_OUT), jnp.float32),
    )(flags, x, wp, bp, wk, bk)
